# baseline (speedup 1.0000x reference)
import jax
import jax.numpy as jnp
from jax.experimental import pallas as pl
from jax.experimental.pallas import tpu as pltpu

_INDEX = 50
_ROWS = 200
_FEAT = 64
_COLS = _ROWS * _FEAT
_COL0 = _INDEX * _FEAT
_TILE1 = _COL0 + 128
_NCH = 4


def _select_scatter_dma(x_ref, src_ref, o_ref, mid, sems):
    b = x_ref.shape[0]
    chunk = b // _NCH
    copies = []
    for c in range(_NCH):
        r = pl.ds(c * chunk, chunk)
        copies.append(pltpu.make_async_copy(
            x_ref.at[r, 0:_COL0], o_ref.at[r, 0:_COL0], sems.at[2 * c]))
        copies.append(pltpu.make_async_copy(
            x_ref.at[r, _TILE1:_COLS], o_ref.at[r, _TILE1:_COLS],
            sems.at[2 * c + 1]))
    mid_in = pltpu.make_async_copy(
        x_ref.at[:, _COL0:_TILE1], mid, sems.at[2 * _NCH])
    for cp in copies:
        cp.start()
    mid_in.start()
    mid_in.wait()
    mid[:, 0:_FEAT] = src_ref[...]
    mid_out = pltpu.make_async_copy(
        mid, o_ref.at[:, _COL0:_TILE1], sems.at[2 * _NCH + 1])
    mid_out.start()
    mid_out.wait()
    for cp in copies:
        cp.wait()


def kernel(x, src):
    b = x.shape[0]
    x2 = x.reshape(b, _COLS)
    out = pl.pallas_call(
        _select_scatter_dma,
        in_specs=[
            pl.BlockSpec(memory_space=pltpu.MemorySpace.HBM),
            pl.BlockSpec(memory_space=pltpu.MemorySpace.VMEM),
        ],
        out_specs=pl.BlockSpec(memory_space=pltpu.MemorySpace.HBM),
        out_shape=jax.ShapeDtypeStruct((b, _COLS), x.dtype),
        scratch_shapes=[
            pltpu.VMEM((b, 128), x.dtype),
            pltpu.SemaphoreType.DMA((2 * _NCH + 2,)),
        ],
    )(x2, src)
    return out.reshape(x.shape)


# speedup vs baseline: 13.1910x; 13.1910x over previous
import jax
import jax.numpy as jnp
from jax.experimental import pallas as pl
from jax.experimental.pallas import tpu as pltpu

_INDEX = 50
_ROWS = 200
_FEAT = 64
_COLS = _ROWS * _FEAT
_COL0 = _INDEX * _FEAT
_BB = 64


def _select_scatter_block(x_ref, src_ref, o_ref):
    o_ref[...] = x_ref[...]
    o_ref[:, _COL0:_COL0 + _FEAT] = src_ref[...]


def kernel(x, src):
    b = x.shape[0]
    x2 = x.reshape(b, _COLS)
    out = pl.pallas_call(
        _select_scatter_block,
        grid=(b // _BB,),
        in_specs=[
            pl.BlockSpec((_BB, _COLS), lambda i: (i, 0)),
            pl.BlockSpec((_BB, _FEAT), lambda i: (i, 0)),
        ],
        out_specs=pl.BlockSpec((_BB, _COLS), lambda i: (i, 0)),
        out_shape=jax.ShapeDtypeStruct((b, _COLS), x.dtype),
        compiler_params=pltpu.CompilerParams(
            dimension_semantics=("parallel",),
        ),
    )(x2, src)
    return out.reshape(x.shape)


# speedup vs baseline: 13.2823x; 1.0069x over previous
import jax
import jax.numpy as jnp
from jax.experimental import pallas as pl
from jax.experimental.pallas import tpu as pltpu

_INDEX = 50
_ROWS = 200
_FEAT = 64
_COLS = _ROWS * _FEAT
_COL0 = _INDEX * _FEAT
_CH = 128
_NBUF = 6
_LEAD = 3


def _select_scatter_pipe(x_ref, src_ref, o_ref, bufs, in_sems, out_sems):
    b = x_ref.shape[0]
    n = b // _CH

    def rows(i):
        return pl.ds(i * _CH, _CH)

    in_copy = [
        pltpu.make_async_copy(x_ref.at[rows(i)], bufs.at[i % _NBUF],
                              in_sems.at[i % _NBUF])
        for i in range(n)
    ]
    out_copy = [
        pltpu.make_async_copy(bufs.at[i % _NBUF], o_ref.at[rows(i)],
                              out_sems.at[i % _NBUF])
        for i in range(n)
    ]

    for i in range(min(_LEAD, n)):
        in_copy[i].start()
    for i in range(n):
        j = i + _LEAD
        if j < n:
            if j >= _NBUF:
                out_copy[j - _NBUF].wait()
            in_copy[j].start()
        in_copy[i].wait()
        buf = bufs.at[i % _NBUF]
        buf[:, _COL0:_COL0 + _FEAT] = src_ref[rows(i), :]
        out_copy[i].start()
    for i in range(max(n - _NBUF, 0), n):
        out_copy[i].wait()


def kernel(x, src):
    b = x.shape[0]
    x2 = x.reshape(b, _COLS)
    out = pl.pallas_call(
        _select_scatter_pipe,
        in_specs=[
            pl.BlockSpec(memory_space=pltpu.MemorySpace.HBM),
            pl.BlockSpec(memory_space=pltpu.MemorySpace.VMEM),
        ],
        out_specs=pl.BlockSpec(memory_space=pltpu.MemorySpace.HBM),
        out_shape=jax.ShapeDtypeStruct((b, _COLS), x.dtype),
        scratch_shapes=[
            pltpu.VMEM((_NBUF, _CH, _COLS), x.dtype),
            pltpu.SemaphoreType.DMA((_NBUF,)),
            pltpu.SemaphoreType.DMA((_NBUF,)),
        ],
    )(x2, src)
    return out.reshape(x.shape)
